# baseline (speedup 1.0000x reference)
import functools

import jax
import jax.numpy as jnp
from jax import lax
from jax.experimental import pallas as pl
from jax.experimental.pallas import tpu as pltpu
from jax.experimental.pallas import tpu_sc as plsc

N = 6_400_000
G = 4096
NC, NS, L = 2, 16, 16
NW = NC * NS
PER_W = N // NW
CH = 10000
NCH = PER_W // CH
VPC = CH // L
UNROLL = 5

_P = (3.373239892967561e-08, 0.9999942784760739, -0.4998386374565373,
      0.3315490396497364, -0.23982739689827703, 0.16582476151536812,
      -0.09325384793721041, 0.03485054365140574, -0.006151618881606863)


def _sc_partials(logits, targets, edge_batch):
    mesh = plsc.VectorSubcoreMesh(core_axis_name="c", subcore_axis_name="s")

    @functools.partial(
        pl.kernel,
        out_type=jax.ShapeDtypeStruct((NW, 4, G), jnp.float32),
        mesh=mesh,
        scratch_types=[
            pltpu.VMEM((CH,), jnp.float32),
            pltpu.VMEM((CH,), jnp.float32),
            pltpu.VMEM((CH,), jnp.float32),
            pltpu.VMEM((CH,), jnp.float32),
            pltpu.VMEM((CH,), jnp.int32),
            pltpu.VMEM((CH,), jnp.int32),
            pltpu.VMEM((2 * G,), jnp.float32),
            pltpu.VMEM((2 * G,), jnp.float32),
            pltpu.VMEM((2 * G,), jnp.float32),
            pltpu.VMEM((2 * G,), jnp.float32),
            pltpu.SemaphoreType.DMA,
            pltpu.SemaphoreType.DMA,
        ],
        compiler_params=pltpu.CompilerParams(needs_layout_passes=False),
    )
    def k(lg_hbm, tg_hbm, eb_hbm, out_hbm, lbuf0, lbuf1, tbuf0, tbuf1,
          sbuf0, sbuf1, a_cnt, a_den, a_num, a_bce, sem0, sem1):
        wid = lax.axis_index("s") * NC + lax.axis_index("c")
        base = wid * PER_W
        slots = ((lbuf0, tbuf0, sbuf0, sem0), (lbuf1, tbuf1, sbuf1, sem1))

        def start(c, slot):
            lb, tb, sb, sem = slot
            off = base + c * CH
            pltpu.make_async_copy(lg_hbm.at[pl.ds(off, CH)], lb, sem).start()
            pltpu.make_async_copy(tg_hbm.at[pl.ds(off, CH)], tb, sem).start()
            pltpu.make_async_copy(eb_hbm.at[pl.ds(off, CH)], sb, sem).start()

        def drain(slot):
            lb, tb, sb, sem = slot
            pltpu.make_async_copy(lg_hbm.at[pl.ds(base, CH)], lb, sem).wait()
            pltpu.make_async_copy(tg_hbm.at[pl.ds(base, CH)], tb, sem).wait()
            pltpu.make_async_copy(eb_hbm.at[pl.ds(base, CH)], sb, sem).wait()

        zeros = jnp.zeros((L,), jnp.float32)

        def zero_body(j, carry):
            off = j * L
            a_cnt[pl.ds(off, L)] = zeros
            a_den[pl.ds(off, L)] = zeros
            a_num[pl.ds(off, L)] = zeros
            a_bce[pl.ds(off, L)] = zeros
            return carry

        lax.fori_loop(0, 2 * G // L, zero_body, 0)

        ones = jnp.ones((L,), jnp.float32)
        lane_off = lax.iota(jnp.int32, L) * VPC
        par_off = (lax.iota(jnp.int32, L) % 2) * G

        def compute(slot):
            lb, tb, sb, _ = slot

            @plsc.parallel_loop(0, VPC, unroll=UNROLL)
            def vec_body(i):
                idx = lane_off + i
                x = plsc.load_gather(lb, [idx])
                t = plsc.load_gather(tb, [idx])
                s = plsc.load_gather(sb, [idx]) + par_off
                w = jnp.minimum(jnp.maximum(t, 0.0), 1.0)
                ex = jnp.exp(x)
                u = jnp.exp(-jnp.abs(x))
                sp = _P[8]
                for cf in _P[7::-1]:
                    sp = sp * u + cf
                bce = jnp.maximum(x, 0.0) - x * t + sp
                plsc.addupdate_scatter(a_cnt, [s], ones)
                plsc.addupdate_scatter(a_den, [s], ex)
                plsc.addupdate_scatter(a_num, [s], ex * w)
                plsc.addupdate_scatter(a_bce, [s], bce)

        start(0, slots[0])

        def pair_body(p, carry):
            c0 = 2 * p
            start(c0 + 1, slots[1])
            drain(slots[0])
            compute(slots[0])
            start(jnp.minimum(c0 + 2, NCH - 1), slots[0])
            drain(slots[1])
            compute(slots[1])
            return carry

        lax.fori_loop(0, NCH // 2, pair_body, 0)
        drain(slots[0])

        def fold_body(j, carry):
            off = j * L
            a_cnt[pl.ds(off, L)] += a_cnt[pl.ds(G + off, L)]
            a_den[pl.ds(off, L)] += a_den[pl.ds(G + off, L)]
            a_num[pl.ds(off, L)] += a_num[pl.ds(G + off, L)]
            a_bce[pl.ds(off, L)] += a_bce[pl.ds(G + off, L)]
            return carry

        lax.fori_loop(0, G // L, fold_body, 0)

        pltpu.sync_copy(a_cnt.at[pl.ds(0, G)], out_hbm.at[wid, 0])
        pltpu.sync_copy(a_den.at[pl.ds(0, G)], out_hbm.at[wid, 1])
        pltpu.sync_copy(a_num.at[pl.ds(0, G)], out_hbm.at[wid, 2])
        pltpu.sync_copy(a_bce.at[pl.ds(0, G)], out_hbm.at[wid, 3])

    return k(logits, targets, edge_batch)


def _finalize_body(p_ref, o_ref):
    acc = jnp.sum(p_ref[...], axis=0)
    cnt = acc[0:1]
    den = acc[1:2]
    num = acc[2:3]
    bces = acc[3:4]
    tiny = jnp.finfo(jnp.float32).tiny
    has_pos = num > 0
    lw = jnp.log(jnp.maximum(den, tiny)) - jnp.log(jnp.maximum(num, tiny))
    n_pos = jnp.maximum(
        jnp.sum(has_pos.astype(jnp.float32), axis=(0, 1), keepdims=True), 1.0)
    listwise = jnp.sum(
        jnp.where(has_pos, lw, 0.0), axis=(0, 1), keepdims=True) / n_pos
    bce_loss = jnp.sum(
        bces / jnp.maximum(cnt, 1.0), axis=(0, 1), keepdims=True) * (1.0 / G)
    o_ref[...] = listwise + 0.5 * bce_loss


def _finalize_tc(partials):
    return pl.pallas_call(
        _finalize_body,
        out_shape=jax.ShapeDtypeStruct((1, 1), jnp.float32),
    )(partials)


def kernel(logits, targets, edge_batch):
    parts = _sc_partials(logits, targets, edge_batch)
    out = _finalize_tc(parts)
    return out.reshape(())


# speedup vs baseline: 1.0201x; 1.0201x over previous
import functools

import jax
import jax.numpy as jnp
from jax import lax
from jax.experimental import pallas as pl
from jax.experimental.pallas import tpu as pltpu
from jax.experimental.pallas import tpu_sc as plsc

N = 6_400_000
G = 4096
NC, NS, L = 2, 16, 16
NW = NC * NS
PER_W = N // NW
CH = 10000
NCH = PER_W // CH
VPC = CH // L
UNROLL = 5

_C3, _C5, _C7, _C9, _C11 = 1 / 3, 1 / 5, 1 / 7, 1 / 9, 1 / 11


def _sc_partials(logits, targets, edge_batch):
    mesh = plsc.VectorSubcoreMesh(core_axis_name="c", subcore_axis_name="s")

    @functools.partial(
        pl.kernel,
        out_type=jax.ShapeDtypeStruct((NW, 4, G), jnp.float32),
        mesh=mesh,
        scratch_types=[
            pltpu.VMEM((CH,), jnp.float32),
            pltpu.VMEM((CH,), jnp.float32),
            pltpu.VMEM((CH,), jnp.float32),
            pltpu.VMEM((CH,), jnp.float32),
            pltpu.VMEM((CH,), jnp.int32),
            pltpu.VMEM((CH,), jnp.int32),
            pltpu.VMEM((2 * G,), jnp.float32),
            pltpu.VMEM((2 * G,), jnp.float32),
            pltpu.VMEM((2 * G,), jnp.float32),
            pltpu.VMEM((2 * G,), jnp.float32),
            pltpu.SemaphoreType.DMA,
            pltpu.SemaphoreType.DMA,
        ],
        compiler_params=pltpu.CompilerParams(needs_layout_passes=False, use_tc_tiling_on_sc=False),
    )
    def k(lg_hbm, tg_hbm, eb_hbm, out_hbm, lbuf0, lbuf1, tbuf0, tbuf1,
          sbuf0, sbuf1, a_cnt, a_den, a_num, a_bce, sem0, sem1):
        wid = lax.axis_index("s") * NC + lax.axis_index("c")
        base = wid * PER_W
        slots = ((lbuf0, tbuf0, sbuf0, sem0), (lbuf1, tbuf1, sbuf1, sem1))

        def start(c, slot):
            lb, tb, sb, sem = slot
            off = base + c * CH
            pltpu.make_async_copy(lg_hbm.at[pl.ds(off, CH)], lb, sem).start()
            pltpu.make_async_copy(tg_hbm.at[pl.ds(off, CH)], tb, sem).start()
            pltpu.make_async_copy(eb_hbm.at[pl.ds(off, CH)], sb, sem).start()

        def drain(slot):
            lb, tb, sb, sem = slot
            pltpu.make_async_copy(lg_hbm.at[pl.ds(base, CH)], lb, sem).wait()
            pltpu.make_async_copy(tg_hbm.at[pl.ds(base, CH)], tb, sem).wait()
            pltpu.make_async_copy(eb_hbm.at[pl.ds(base, CH)], sb, sem).wait()

        zeros = jnp.zeros((L,), jnp.float32)

        def zero_body(j, carry):
            off = j * L
            a_cnt[pl.ds(off, L)] = zeros
            a_den[pl.ds(off, L)] = zeros
            a_num[pl.ds(off, L)] = zeros
            a_bce[pl.ds(off, L)] = zeros
            return carry

        lax.fori_loop(0, 2 * G // L, zero_body, 0)

        ones = jnp.ones((L,), jnp.float32)
        lane_off = lax.iota(jnp.int32, L) * VPC
        par_off = (lax.iota(jnp.int32, L) % 2) * G

        def compute(slot):
            lb, tb, sb, _ = slot

            @plsc.parallel_loop(0, VPC, unroll=UNROLL)
            def vec_body(i):
                idx = lane_off + i
                x = plsc.load_gather(lb, [idx])
                t = plsc.load_gather(tb, [idx])
                s = plsc.load_gather(sb, [idx]) + par_off
                w = jnp.minimum(jnp.maximum(t, 0.0), 1.0)
                ex = jnp.exp(x)
                u = jnp.exp(-jnp.abs(x))
                z = u / (u + 2.0)
                z2 = z * z
                p = z * (1.0 + z2 * (_C3 + z2 * (_C5 + z2 * (_C7 + z2 * (_C9 + z2 * _C11)))))
                bce = jnp.maximum(x, 0.0) - x * t + 2.0 * p
                plsc.addupdate_scatter(a_cnt, [s], ones)
                plsc.addupdate_scatter(a_den, [s], ex)
                plsc.addupdate_scatter(a_num, [s], ex * w)
                plsc.addupdate_scatter(a_bce, [s], bce)

        start(0, slots[0])

        def pair_body(p, carry):
            c0 = 2 * p
            start(c0 + 1, slots[1])
            drain(slots[0])
            compute(slots[0])
            start(jnp.minimum(c0 + 2, NCH - 1), slots[0])
            drain(slots[1])
            compute(slots[1])
            return carry

        lax.fori_loop(0, NCH // 2, pair_body, 0)
        drain(slots[0])

        def fold_body(j, carry):
            off = j * L
            a_cnt[pl.ds(off, L)] += a_cnt[pl.ds(G + off, L)]
            a_den[pl.ds(off, L)] += a_den[pl.ds(G + off, L)]
            a_num[pl.ds(off, L)] += a_num[pl.ds(G + off, L)]
            a_bce[pl.ds(off, L)] += a_bce[pl.ds(G + off, L)]
            return carry

        lax.fori_loop(0, G // L, fold_body, 0)

        pltpu.sync_copy(a_cnt.at[pl.ds(0, G)], out_hbm.at[wid, 0])
        pltpu.sync_copy(a_den.at[pl.ds(0, G)], out_hbm.at[wid, 1])
        pltpu.sync_copy(a_num.at[pl.ds(0, G)], out_hbm.at[wid, 2])
        pltpu.sync_copy(a_bce.at[pl.ds(0, G)], out_hbm.at[wid, 3])

    return k(logits, targets, edge_batch)


def _finalize_body(p_ref, o_ref):
    acc = jnp.sum(p_ref[...], axis=0)
    cnt = acc[0:1]
    den = acc[1:2]
    num = acc[2:3]
    bces = acc[3:4]
    tiny = jnp.finfo(jnp.float32).tiny
    has_pos = num > 0
    lw = jnp.log(jnp.maximum(den, tiny)) - jnp.log(jnp.maximum(num, tiny))
    n_pos = jnp.maximum(
        jnp.sum(has_pos.astype(jnp.float32), axis=(0, 1), keepdims=True), 1.0)
    listwise = jnp.sum(
        jnp.where(has_pos, lw, 0.0), axis=(0, 1), keepdims=True) / n_pos
    bce_loss = jnp.sum(
        bces / jnp.maximum(cnt, 1.0), axis=(0, 1), keepdims=True) * (1.0 / G)
    o_ref[...] = listwise + 0.5 * bce_loss


def _finalize_tc(partials):
    return pl.pallas_call(
        _finalize_body,
        out_shape=jax.ShapeDtypeStruct((1, 1), jnp.float32),
    )(partials)


def kernel(logits, targets, edge_batch):
    parts = _sc_partials(logits, targets, edge_batch)
    out = _finalize_tc(parts)
    return out.reshape(())


# speedup vs baseline: 1.0637x; 1.0427x over previous
import functools

import jax
import jax.numpy as jnp
from jax import lax
from jax.experimental import pallas as pl
from jax.experimental.pallas import tpu as pltpu
from jax.experimental.pallas import tpu_sc as plsc

N = 6_400_000
G = 4096
NC, NS, L = 2, 16, 16
NW = NC * NS
PER_W = N // NW
CH = 10000
NCH = PER_W // CH
VPC = CH // L
UNROLL = 5

_C3, _C5, _C7, _C9, _C11 = 1 / 3, 1 / 5, 1 / 7, 1 / 9, 1 / 11


def _sc_partials(logits, targets, edge_batch):
    mesh = plsc.VectorSubcoreMesh(core_axis_name="c", subcore_axis_name="s")

    @functools.partial(
        pl.kernel,
        out_type=jax.ShapeDtypeStruct((NW, 4, G), jnp.float32),
        mesh=mesh,
        scratch_types=[
            pltpu.VMEM((CH,), jnp.float32),
            pltpu.VMEM((CH,), jnp.float32),
            pltpu.VMEM((CH,), jnp.float32),
            pltpu.VMEM((CH,), jnp.float32),
            pltpu.VMEM((CH,), jnp.int32),
            pltpu.VMEM((CH,), jnp.int32),
            pltpu.VMEM((2 * G,), jnp.float32),
            pltpu.VMEM((2 * G,), jnp.float32),
            pltpu.VMEM((2 * G,), jnp.float32),
            pltpu.VMEM((2 * G,), jnp.float32),
            pltpu.SemaphoreType.DMA,
            pltpu.SemaphoreType.DMA,
        ],
        compiler_params=pltpu.CompilerParams(needs_layout_passes=False),
    )
    def k(lg_hbm, tg_hbm, eb_hbm, out_hbm, lbuf0, lbuf1, tbuf0, tbuf1,
          sbuf0, sbuf1, a_cnt, a_den, a_num, a_bce, sem0, sem1):
        wid = lax.axis_index("s") * NC + lax.axis_index("c")
        base = wid * PER_W
        slots = ((lbuf0, tbuf0, sbuf0, sem0), (lbuf1, tbuf1, sbuf1, sem1))

        def start(c, slot):
            lb, tb, sb, sem = slot
            off = base + c * CH
            pltpu.make_async_copy(lg_hbm.at[pl.ds(off, CH)], lb, sem).start()
            pltpu.make_async_copy(tg_hbm.at[pl.ds(off, CH)], tb, sem).start()
            pltpu.make_async_copy(eb_hbm.at[pl.ds(off, CH)], sb, sem).start()

        def drain(slot):
            lb, tb, sb, sem = slot
            pltpu.make_async_copy(lg_hbm.at[pl.ds(base, CH)], lb, sem).wait()
            pltpu.make_async_copy(tg_hbm.at[pl.ds(base, CH)], tb, sem).wait()
            pltpu.make_async_copy(eb_hbm.at[pl.ds(base, CH)], sb, sem).wait()

        start(0, slots[0])

        zeros = jnp.zeros((L,), jnp.float32)

        def zero_body(j, carry):
            off = j * L
            a_cnt[pl.ds(off, L)] = zeros
            a_den[pl.ds(off, L)] = zeros
            a_num[pl.ds(off, L)] = zeros
            a_bce[pl.ds(off, L)] = zeros
            return carry

        lax.fori_loop(0, 2 * G // L, zero_body, 0)
        ones = jnp.ones((L,), jnp.float32)
        lane_off = lax.iota(jnp.int32, L) * VPC
        par_off = (lax.iota(jnp.int32, L) % 2) * G

        def compute(slot):
            lb, tb, sb, _ = slot

            @plsc.parallel_loop(0, VPC, unroll=UNROLL)
            def vec_body(i):
                idx = lane_off + i
                x = plsc.load_gather(lb, [idx])
                t = plsc.load_gather(tb, [idx])
                s = plsc.load_gather(sb, [idx]) + par_off
                w = jnp.minimum(jnp.maximum(t, 0.0), 1.0)
                ex = jnp.exp(x)
                u = jnp.exp(-jnp.abs(x))
                z = u / (u + 2.0)
                z2 = z * z
                p = z * (1.0 + z2 * (_C3 + z2 * (_C5 + z2 * (_C7 + z2 * (_C9 + z2 * _C11)))))
                bce = jnp.maximum(x, 0.0) - x * t + 2.0 * p
                plsc.addupdate_scatter(a_cnt, [s], ones)
                plsc.addupdate_scatter(a_den, [s], ex)
                plsc.addupdate_scatter(a_num, [s], ex * w)
                plsc.addupdate_scatter(a_bce, [s], bce)

        def pair_body(p, carry):
            c0 = 2 * p
            start(c0 + 1, slots[1])
            drain(slots[0])
            compute(slots[0])
            @pl.when(c0 + 2 < NCH)
            def _():
                start(c0 + 2, slots[0])

            drain(slots[1])
            compute(slots[1])
            return carry

        lax.fori_loop(0, NCH // 2, pair_body, 0)

        def fold_body(j, carry):
            off = j * L
            a_cnt[pl.ds(off, L)] += a_cnt[pl.ds(G + off, L)]
            a_den[pl.ds(off, L)] += a_den[pl.ds(G + off, L)]
            a_num[pl.ds(off, L)] += a_num[pl.ds(G + off, L)]
            a_bce[pl.ds(off, L)] += a_bce[pl.ds(G + off, L)]
            return carry

        lax.fori_loop(0, G // L, fold_body, 0)

        pltpu.sync_copy(a_cnt.at[pl.ds(0, G)], out_hbm.at[wid, 0])
        pltpu.sync_copy(a_den.at[pl.ds(0, G)], out_hbm.at[wid, 1])
        pltpu.sync_copy(a_num.at[pl.ds(0, G)], out_hbm.at[wid, 2])
        pltpu.sync_copy(a_bce.at[pl.ds(0, G)], out_hbm.at[wid, 3])

    return k(logits, targets, edge_batch)


def _finalize_body(p_ref, o_ref):
    acc = jnp.sum(p_ref[...], axis=0)
    cnt = acc[0:1]
    den = acc[1:2]
    num = acc[2:3]
    bces = acc[3:4]
    tiny = jnp.finfo(jnp.float32).tiny
    has_pos = num > 0
    lw = jnp.log(jnp.maximum(den, tiny)) - jnp.log(jnp.maximum(num, tiny))
    n_pos = jnp.maximum(
        jnp.sum(has_pos.astype(jnp.float32), axis=(0, 1), keepdims=True), 1.0)
    listwise = jnp.sum(
        jnp.where(has_pos, lw, 0.0), axis=(0, 1), keepdims=True) / n_pos
    bce_loss = jnp.sum(
        bces / jnp.maximum(cnt, 1.0), axis=(0, 1), keepdims=True) * (1.0 / G)
    o_ref[...] = listwise + 0.5 * bce_loss


def _finalize_tc(partials):
    return pl.pallas_call(
        _finalize_body,
        out_shape=jax.ShapeDtypeStruct((1, 1), jnp.float32),
    )(partials)


def kernel(logits, targets, edge_batch):
    parts = _sc_partials(logits, targets, edge_batch)
    out = _finalize_tc(parts)
    return out.reshape(())


# speedup vs baseline: 1.0828x; 1.0180x over previous
import functools

import jax
import jax.numpy as jnp
from jax import lax
from jax.experimental import pallas as pl
from jax.experimental.pallas import tpu as pltpu
from jax.experimental.pallas import tpu_sc as plsc

N = 6_400_000
G = 4096
NC, NS, L = 2, 16, 16
NW = NC * NS
PER_W = N // NW
CH = 10000
NCH = PER_W // CH
VPC = CH // L
UNROLL = 5

_C3, _C5, _C7, _C9, _C11 = 1 / 3, 1 / 5, 1 / 7, 1 / 9, 1 / 11


def _sc_partials(logits, targets, edge_batch):
    mesh = plsc.VectorSubcoreMesh(core_axis_name="c", subcore_axis_name="s")

    @functools.partial(
        pl.kernel,
        out_type=jax.ShapeDtypeStruct((NW, 4, G), jnp.float32),
        mesh=mesh,
        scratch_types=[
            pltpu.VMEM((CH,), jnp.float32),
            pltpu.VMEM((CH,), jnp.float32),
            pltpu.VMEM((CH,), jnp.float32),
            pltpu.VMEM((CH,), jnp.float32),
            pltpu.VMEM((CH,), jnp.int32),
            pltpu.VMEM((CH,), jnp.int32),
            pltpu.VMEM((2 * G,), jnp.float32),
            pltpu.VMEM((2 * G,), jnp.float32),
            pltpu.VMEM((2 * G,), jnp.float32),
            pltpu.VMEM((2 * G,), jnp.float32),
            pltpu.SemaphoreType.DMA,
            pltpu.SemaphoreType.DMA,
        ],
        compiler_params=pltpu.CompilerParams(needs_layout_passes=False),
    )
    def k(lg_hbm, tg_hbm, eb_hbm, out_hbm, lbuf0, lbuf1, tbuf0, tbuf1,
          sbuf0, sbuf1, a_cnt, a_den, a_num, a_bce, sem0, sem1):
        wid = lax.axis_index("s") * NC + lax.axis_index("c")
        base = wid * PER_W
        slots = ((lbuf0, tbuf0, sbuf0, sem0), (lbuf1, tbuf1, sbuf1, sem1))

        def start(c, slot):
            lb, tb, sb, sem = slot
            off = base + c * CH
            pltpu.make_async_copy(lg_hbm.at[pl.ds(off, CH)], lb, sem).start()
            pltpu.make_async_copy(tg_hbm.at[pl.ds(off, CH)], tb, sem).start()
            pltpu.make_async_copy(eb_hbm.at[pl.ds(off, CH)], sb, sem).start()

        def drain(slot):
            lb, tb, sb, sem = slot
            pltpu.make_async_copy(lg_hbm.at[pl.ds(base, CH)], lb, sem).wait()
            pltpu.make_async_copy(tg_hbm.at[pl.ds(base, CH)], tb, sem).wait()
            pltpu.make_async_copy(eb_hbm.at[pl.ds(base, CH)], sb, sem).wait()

        start(0, slots[0])

        zeros = jnp.zeros((L,), jnp.float32)

        def zero_body(j, carry):
            off = j * L
            a_cnt[pl.ds(off, L)] = zeros
            a_den[pl.ds(off, L)] = zeros
            a_num[pl.ds(off, L)] = zeros
            a_bce[pl.ds(off, L)] = zeros
            return carry

        lax.fori_loop(0, 2 * G // L, zero_body, 0)
        ones = jnp.ones((L,), jnp.float32)
        lane_off = lax.iota(jnp.int32, L) * VPC
        lane_iota = lax.iota(jnp.int32, L)

        def compute(slot):
            lb, tb, sb, _ = slot

            @plsc.parallel_loop(0, VPC, unroll=UNROLL)
            def vec_body(i):
                idx = lane_off + i
                x = plsc.load_gather(lb, [idx])
                t = plsc.load_gather(tb, [idx])
                par = ((lane_iota + i) & 1) * G
                s = plsc.load_gather(sb, [idx]) + par
                w = jnp.minimum(jnp.maximum(t, 0.0), 1.0)
                ex = jnp.exp(x)
                u = jnp.exp(-jnp.abs(x))
                z = u / (u + 2.0)
                z2 = z * z
                p = z * (1.0 + z2 * (_C3 + z2 * (_C5 + z2 * (_C7 + z2 * (_C9 + z2 * _C11)))))
                bce = jnp.maximum(x, 0.0) - x * t + 2.0 * p
                plsc.addupdate_scatter(a_cnt, [s], ones)
                plsc.addupdate_scatter(a_den, [s], ex)
                plsc.addupdate_scatter(a_num, [s], ex * w)
                plsc.addupdate_scatter(a_bce, [s], bce)

        def pair_body(p, carry):
            c0 = 2 * p
            start(c0 + 1, slots[1])
            drain(slots[0])
            compute(slots[0])
            @pl.when(c0 + 2 < NCH)
            def _():
                start(c0 + 2, slots[0])

            drain(slots[1])
            compute(slots[1])
            return carry

        lax.fori_loop(0, NCH // 2, pair_body, 0)

        def fold_body(j, carry):
            off = j * L
            a_cnt[pl.ds(off, L)] += a_cnt[pl.ds(G + off, L)]
            a_den[pl.ds(off, L)] += a_den[pl.ds(G + off, L)]
            a_num[pl.ds(off, L)] += a_num[pl.ds(G + off, L)]
            a_bce[pl.ds(off, L)] += a_bce[pl.ds(G + off, L)]
            return carry

        lax.fori_loop(0, G // L, fold_body, 0)

        pltpu.sync_copy(a_cnt.at[pl.ds(0, G)], out_hbm.at[wid, 0])
        pltpu.sync_copy(a_den.at[pl.ds(0, G)], out_hbm.at[wid, 1])
        pltpu.sync_copy(a_num.at[pl.ds(0, G)], out_hbm.at[wid, 2])
        pltpu.sync_copy(a_bce.at[pl.ds(0, G)], out_hbm.at[wid, 3])

    return k(logits, targets, edge_batch)


def _finalize_body(p_ref, o_ref):
    acc = jnp.sum(p_ref[...], axis=0)
    cnt = acc[0:1]
    den = acc[1:2]
    num = acc[2:3]
    bces = acc[3:4]
    tiny = jnp.finfo(jnp.float32).tiny
    has_pos = num > 0
    lw = jnp.log(jnp.maximum(den, tiny)) - jnp.log(jnp.maximum(num, tiny))
    n_pos = jnp.maximum(
        jnp.sum(has_pos.astype(jnp.float32), axis=(0, 1), keepdims=True), 1.0)
    listwise = jnp.sum(
        jnp.where(has_pos, lw, 0.0), axis=(0, 1), keepdims=True) / n_pos
    bce_loss = jnp.sum(
        bces / jnp.maximum(cnt, 1.0), axis=(0, 1), keepdims=True) * (1.0 / G)
    o_ref[...] = listwise + 0.5 * bce_loss


def _finalize_tc(partials):
    return pl.pallas_call(
        _finalize_body,
        out_shape=jax.ShapeDtypeStruct((1, 1), jnp.float32),
    )(partials)


def kernel(logits, targets, edge_batch):
    parts = _sc_partials(logits, targets, edge_batch)
    out = _finalize_tc(parts)
    return out.reshape(())
